# baseline (speedup 1.0000x reference)
import functools

import jax
import jax.numpy as jnp
from jax import lax
from jax.experimental import pallas as pl
from jax.experimental.pallas import tpu as pltpu
from jax.experimental.pallas import tpu_sc as plsc

B, S, D, E, TOP_K = 4, 4096, 4096, 64, 8
T = B * S

BT = 1024
DC = 4096
ND = D // DC
CHUNKS = 2
TC_ = T // CHUNKS

NC, NS, L = 2, 16, 16
NW = NC * NS
TOK_W = TC_ // NW
NG = TOK_W // L


def _logits_body(x_ref, m_ref, w_ref, b_ref, lg_ref):
    d = pl.program_id(1)
    part = lax.dot_general(
        w_ref[...], x_ref[...],
        dimension_numbers=(((1,), (1,)), ((), ())),
        preferred_element_type=jnp.float32,
    )

    @pl.when(d == 0)
    def _():
        lg_ref[...] = part + b_ref[...]

    @pl.when(d > 0)
    def _():
        acc = lg_ref[...] + part
        if ND > 1:
            lg_ref[...] = jnp.where(
                (d == ND - 1) & (m_ref[...] != 1), -jnp.inf, acc
            )

    if ND == 1:
        lg_ref[...] = jnp.where(m_ref[...] != 1, -jnp.inf, lg_ref[...])


def _route_body(lg_hbm, ew_hbm, ei_hbm, lg_v, ew_v, ei_v, sem):
    wid = lax.axis_index("s") * NC + lax.axis_index("c")
    base = wid * TOK_W
    pltpu.sync_copy(lg_hbm.at[:, pl.ds(base, TOK_W)], lg_v)

    neg_inf = jnp.full((L,), -jnp.inf, jnp.float32)

    def group(g, _):
        topv = [neg_inf] * TOP_K
        topi = [jnp.zeros((L,), jnp.int32)] * TOP_K
        for e in range(E):
            xv = lg_v[e, pl.ds(g * L, L)]
            xi = jnp.full((L,), e, jnp.int32)
            for j in range(TOP_K):
                c = xv > topv[j]
                nv = jnp.where(c, xv, topv[j])
                xv = jnp.where(c, topv[j], xv)
                ni = jnp.where(c, xi, topi[j])
                xi = jnp.where(c, topi[j], xi)
                topv[j] = nv
                topi[j] = ni
        es = [jnp.exp(v - topv[0]) for v in topv]
        tot = es[0]
        for v in es[1:]:
            tot = tot + v
        for j in range(TOP_K):
            ew_v[j, pl.ds(g * L, L)] = es[j] / tot
            ei_v[j, pl.ds(g * L, L)] = topi[j]
        return 0

    lax.fori_loop(0, NG, group, 0)
    pltpu.sync_copy(ew_v, ew_hbm.at[:, pl.ds(base, TOK_W)])
    pltpu.sync_copy(ei_v, ei_hbm.at[:, pl.ds(base, TOK_W)])


_route = functools.partial(
    pl.kernel,
    out_type=[
        jax.ShapeDtypeStruct((TOP_K, TC_), jnp.float32),
        jax.ShapeDtypeStruct((TOP_K, TC_), jnp.int32),
    ],
    mesh=plsc.VectorSubcoreMesh(core_axis_name="c", subcore_axis_name="s"),
    scratch_types=[
        pltpu.VMEM((E, TOK_W), jnp.float32),
        pltpu.VMEM((TOP_K, TOK_W), jnp.float32),
        pltpu.VMEM((TOP_K, TOK_W), jnp.int32),
        pltpu.SemaphoreType.DMA,
    ],
)(_route_body)


@jax.jit
def kernel(x, attention_mask, W, b):
    x2 = x.reshape(T, D)
    m2 = attention_mask.reshape(1, T)
    b2 = b.reshape(E, 1)

    nblk = TC_ // BT
    ews, eis = [], []
    for c in range(CHUNKS):
        logits = pl.pallas_call(
            _logits_body,
            grid=(nblk, ND),
            in_specs=[
                pl.BlockSpec((BT, DC), lambda i, d, c=c: (c * nblk + i, d)),
                pl.BlockSpec((1, BT), lambda i, d, c=c: (0, c * nblk + i)),
                pl.BlockSpec((E, DC), lambda i, d: (0, d)),
                pl.BlockSpec((E, 1), lambda i, d: (0, 0)),
            ],
            out_specs=pl.BlockSpec((E, BT), lambda i, d: (0, i)),
            out_shape=jax.ShapeDtypeStruct((E, TC_), jnp.float32),
        )(x2, m2, W, b2)
        ew_c, ei_c = _route(logits)
        ews.append(ew_c)
        eis.append(ei_c)

    ew = jnp.concatenate(ews, axis=1)
    ei = jnp.concatenate(eis, axis=1)
    return (
        ew.T.reshape(B, S, TOP_K),
        ei.T.reshape(B, S, TOP_K),
    )


# speedup vs baseline: 1.0451x; 1.0451x over previous
import functools

import jax
import jax.numpy as jnp
from jax import lax
from jax.experimental import pallas as pl
from jax.experimental.pallas import tpu as pltpu
from jax.experimental.pallas import tpu_sc as plsc

B, S, D, E, TOP_K = 4, 4096, 4096, 64, 8
T = B * S

BT = 1024
CHUNK_TOKENS = (12288, 4096)

NC, NS, L = 2, 16, 16
NW = NC * NS


def _logits_body(x_ref, m_ref, w_ref, b_ref, lg_ref):
    lg = lax.dot_general(
        w_ref[...], x_ref[...],
        dimension_numbers=(((1,), (1,)), ((), ())),
        preferred_element_type=jnp.float32,
    )
    lg = lg + b_ref[...]
    lg_ref[...] = jnp.where(m_ref[...] != 1, -jnp.inf, lg)


def _make_route(tc):
    tok_w = tc // NW
    ng = tok_w // L

    def _route_body(lg_hbm, ew_hbm, ei_hbm, lg_v, ew_v, ei_v, sem):
        wid = lax.axis_index("s") * NC + lax.axis_index("c")
        base = wid * tok_w
        pltpu.sync_copy(lg_hbm.at[:, pl.ds(base, tok_w)], lg_v)

        neg_inf = jnp.full((L,), -jnp.inf, jnp.float32)

        def group(g, _):
            topv = [neg_inf] * TOP_K
            topi = [jnp.zeros((L,), jnp.int32)] * TOP_K
            for e in range(E):
                xv = lg_v[e, pl.ds(g * L, L)]
                xi = jnp.full((L,), e, jnp.int32)
                for j in range(TOP_K):
                    c = xv > topv[j]
                    nv = jnp.where(c, xv, topv[j])
                    xv = jnp.where(c, topv[j], xv)
                    ni = jnp.where(c, xi, topi[j])
                    xi = jnp.where(c, topi[j], xi)
                    topv[j] = nv
                    topi[j] = ni
            es = [jnp.exp(v - topv[0]) for v in topv]
            tot = es[0]
            for v in es[1:]:
                tot = tot + v
            for j in range(TOP_K):
                ew_v[j, pl.ds(g * L, L)] = es[j] / tot
                ei_v[j, pl.ds(g * L, L)] = topi[j]
            return 0

        lax.fori_loop(0, ng, group, 0)
        pltpu.sync_copy(ew_v, ew_hbm.at[:, pl.ds(base, tok_w)])
        pltpu.sync_copy(ei_v, ei_hbm.at[:, pl.ds(base, tok_w)])

    return functools.partial(
        pl.kernel,
        out_type=[
            jax.ShapeDtypeStruct((TOP_K, tc), jnp.float32),
            jax.ShapeDtypeStruct((TOP_K, tc), jnp.int32),
        ],
        mesh=plsc.VectorSubcoreMesh(core_axis_name="c", subcore_axis_name="s"),
        scratch_types=[
            pltpu.VMEM((E, tok_w), jnp.float32),
            pltpu.VMEM((TOP_K, tok_w), jnp.float32),
            pltpu.VMEM((TOP_K, tok_w), jnp.int32),
            pltpu.SemaphoreType.DMA,
        ],
    )(_route_body)


_routes = {tc: _make_route(tc) for tc in set(CHUNK_TOKENS)}


@jax.jit
def kernel(x, attention_mask, W, b):
    x2 = x.reshape(T, D)
    m2 = attention_mask.reshape(1, T)
    b2 = b.reshape(E, 1)

    off = 0
    ews, eis = [], []
    for tc in CHUNK_TOKENS:
        nblk = tc // BT
        blk0 = off // BT
        logits = pl.pallas_call(
            _logits_body,
            grid=(nblk,),
            in_specs=[
                pl.BlockSpec((BT, D), lambda i, blk0=blk0: (blk0 + i, 0)),
                pl.BlockSpec((1, BT), lambda i, blk0=blk0: (0, blk0 + i)),
                pl.BlockSpec((E, D), lambda i: (0, 0)),
                pl.BlockSpec((E, 1), lambda i: (0, 0)),
            ],
            out_specs=pl.BlockSpec((E, BT), lambda i: (0, i)),
            out_shape=jax.ShapeDtypeStruct((E, tc), jnp.float32),
        )(x2, m2, W, b2)
        ew_c, ei_c = _routes[tc](logits)
        ews.append(ew_c)
        eis.append(ei_c)
        off += tc

    ew = jnp.concatenate(ews, axis=1)
    ei = jnp.concatenate(eis, axis=1)
    return (
        ew.T.reshape(B, S, TOP_K),
        ei.T.reshape(B, S, TOP_K),
    )


# speedup vs baseline: 1.0522x; 1.0067x over previous
import functools

import jax
import jax.numpy as jnp
from jax import lax
from jax.experimental import pallas as pl
from jax.experimental.pallas import tpu as pltpu
from jax.experimental.pallas import tpu_sc as plsc

B, S, D, E, TOP_K = 4, 4096, 4096, 64, 8
T = B * S

BT = 1024
CHUNK_TOKENS = (12288, 4096)

NC, NS, L = 2, 16, 16
NW = NC * NS


def _logits_body(x_ref, m_ref, w_ref, b_ref, lg_ref):
    lg = lax.dot_general(
        w_ref[...], x_ref[...],
        dimension_numbers=(((1,), (1,)), ((), ())),
        preferred_element_type=jnp.float32,
    )
    lg = lg + b_ref[...]
    lg_ref[...] = jnp.where(m_ref[...] != 1, -jnp.inf, lg)


def _make_route(tc):
    tok_w = tc // NW
    ng = tok_w // L

    def _route_body(lg_hbm, ew_hbm, ei_hbm, lg_v, ew_v, ei_v, sem):
        wid = lax.axis_index("s") * NC + lax.axis_index("c")
        base = wid * tok_w
        pltpu.sync_copy(lg_hbm.at[:, pl.ds(base, tok_w)], lg_v)

        neg_inf = jnp.full((L,), -jnp.inf, jnp.float32)

        def group(g, _):
            def expert(e, carry):
                topv = list(carry[:TOP_K])
                topi = list(carry[TOP_K:])
                xv = lg_v[e, pl.ds(g * L, L)]
                xi = jnp.full((L,), e, jnp.int32)
                for j in range(TOP_K):
                    c = xv > topv[j]
                    nv = jnp.where(c, xv, topv[j])
                    xv = jnp.where(c, topv[j], xv)
                    ni = jnp.where(c, xi, topi[j])
                    xi = jnp.where(c, topi[j], xi)
                    topv[j] = nv
                    topi[j] = ni
                return tuple(topv) + tuple(topi)

            init = (neg_inf,) * TOP_K + (jnp.zeros((L,), jnp.int32),) * TOP_K
            carry = lax.fori_loop(0, E, expert, init)
            topv = list(carry[:TOP_K])
            topi = list(carry[TOP_K:])
            es = [jnp.exp(v - topv[0]) for v in topv]
            tot = es[0]
            for v in es[1:]:
                tot = tot + v
            for j in range(TOP_K):
                ew_v[j, pl.ds(g * L, L)] = es[j] / tot
                ei_v[j, pl.ds(g * L, L)] = topi[j]
            return 0

        lax.fori_loop(0, ng, group, 0)
        pltpu.sync_copy(ew_v, ew_hbm.at[:, pl.ds(base, tok_w)])
        pltpu.sync_copy(ei_v, ei_hbm.at[:, pl.ds(base, tok_w)])

    return functools.partial(
        pl.kernel,
        out_type=[
            jax.ShapeDtypeStruct((TOP_K, tc), jnp.float32),
            jax.ShapeDtypeStruct((TOP_K, tc), jnp.int32),
        ],
        mesh=plsc.VectorSubcoreMesh(core_axis_name="c", subcore_axis_name="s"),
        scratch_types=[
            pltpu.VMEM((E, tok_w), jnp.float32),
            pltpu.VMEM((TOP_K, tok_w), jnp.float32),
            pltpu.VMEM((TOP_K, tok_w), jnp.int32),
            pltpu.SemaphoreType.DMA,
        ],
    )(_route_body)


_routes = {tc: _make_route(tc) for tc in set(CHUNK_TOKENS)}


@jax.jit
def kernel(x, attention_mask, W, b):
    x2 = x.reshape(T, D)
    m2 = attention_mask.reshape(1, T)
    b2 = b.reshape(E, 1)

    off = 0
    ews, eis = [], []
    for tc in CHUNK_TOKENS:
        nblk = tc // BT
        blk0 = off // BT
        logits = pl.pallas_call(
            _logits_body,
            grid=(nblk,),
            in_specs=[
                pl.BlockSpec((BT, D), lambda i, blk0=blk0: (blk0 + i, 0)),
                pl.BlockSpec((1, BT), lambda i, blk0=blk0: (0, blk0 + i)),
                pl.BlockSpec((E, D), lambda i: (0, 0)),
                pl.BlockSpec((E, 1), lambda i: (0, 0)),
            ],
            out_specs=pl.BlockSpec((E, BT), lambda i: (0, i)),
            out_shape=jax.ShapeDtypeStruct((E, tc), jnp.float32),
        )(x2, m2, W, b2)
        ew_c, ei_c = _routes[tc](logits)
        ews.append(ew_c)
        eis.append(ei_c)
        off += tc

    ew = jnp.concatenate(ews, axis=1)
    ei = jnp.concatenate(eis, axis=1)
    return (
        ew.T.reshape(B, S, TOP_K),
        ei.T.reshape(B, S, TOP_K),
    )
